# baseline (device time: 105671 ns/iter reference)
import jax
import jax.numpy as jnp
from jax import lax
from jax.experimental import pallas as pl
from jax.experimental.pallas import tpu as pltpu

N_DEV = 8
M = 3072
M_CHUNK = M // N_DEV
N_THIRDS = 3

A_STEPS = 16
B_STEPS = 16


def kernel(A, B):
    m, k = A.shape
    k2, n = B.shape
    assert m == M and k == k2
    nw = n // N_THIRDS
    a_rows = M_CHUNK // 2
    b_rows = k // B_STEPS

    def body(a_hbm, b_hbm, out_hbm, a16_ref, b16_ref, sta_ref, stb_ref,
             sb1_ref, rb1_ref, w_ref, rb2_ref, rb3_ref, ob_ref,
             cp_sems_a, cp_sems_b, out_sems,
             s1, r1, s1b, r1b, s2, r2, s2b, r2b, s3, r3, exit_sem):
        my = lax.axis_index("i")

        def fx(p):
            return p + 1 - 2 * lax.rem(p, 2)

        def fy(p):
            p4 = lax.rem(p, 4)
            return p - p4 + 3 - p4

        def fz(p):
            return lax.rem(p + 4, N_DEV)

        def F(p, mask):
            bx = mask // 4
            by = lax.rem(mask, 4) // 2
            bz = lax.rem(mask, 2)
            p = bz * fz(p) + (1 - bz) * p
            p = by * fy(p) + (1 - by) * p
            p = bx * fx(p) + (1 - bx) * p
            return p

        axes = [(fx, fy, fz), (fy, fz, fx), (fz, fx, fy)]

        def w_chunk(g, mi):
            _, F1, F2 = axes[g]
            c = my
            if mi & 1:
                c = F2(c)
            if mi & 2:
                c = F1(c)
            return c

        def out_chunk(g, mi):
            return axes[g][0](w_chunk(g, mi))

        barrier_sem = pltpu.get_barrier_semaphore()
        for g in range(N_THIRDS):
            pl.semaphore_signal(
                barrier_sem, inc=1,
                device_id=(axes[g][0](my),),
                device_id_type=pl.DeviceIdType.MESH,
            )
        pl.semaphore_wait(barrier_sem, 3)

        def b_dma(j):
            return pltpu.make_async_copy(
                b_hbm.at[pl.ds(j * b_rows, b_rows), :],
                stb_ref.at[lax.rem(j, 2)], cp_sems_b.at[lax.rem(j, 2)],
            )

        def a_chunk_of(t):
            return F(my, 7 - t // 2)

        def a_row(t):
            return a_chunk_of(t) * M_CHUNK + lax.rem(t, 2) * a_rows

        def a_dma(t):
            return pltpu.make_async_copy(
                a_hbm.at[pl.ds(a_row(t), a_rows), :],
                sta_ref.at[lax.rem(t, 2)], cp_sems_a.at[lax.rem(t, 2)],
            )

        b_dma(0).start()
        b_dma(1).start()
        a_dma(0).start()
        a_dma(1).start()

        def b_body(j, _):
            sl = lax.rem(j, 2)
            b_dma(j).wait()
            b16_ref[pl.ds(j * b_rows, b_rows), :] = (
                stb_ref[sl].astype(jnp.bfloat16))

            @pl.when(j + 2 < B_STEPS)
            def _():
                b_dma(j + 2).start()
            return 0

        lax.fori_loop(0, B_STEPS, b_body, 0)

        def rows(c):
            return pl.ds(c * M_CHUNK, M_CHUNK)

        def pp16(c, g):
            return jnp.dot(
                a16_ref[rows(c), :], b16_ref[:, g * nw:(g + 1) * nw],
                preferred_element_type=jnp.float32,
            ).astype(jnp.bfloat16)

        def rdma1a(g):
            return pltpu.make_async_remote_copy(
                src_ref=sb1_ref.at[g, pl.ds(2, 2)],
                dst_ref=rb1_ref.at[g, pl.ds(2, 2)],
                send_sem=s1.at[g], recv_sem=r1.at[g],
                device_id=(axes[g][0](my),),
                device_id_type=pl.DeviceIdType.MESH,
            )

        def rdma1b(g):
            return pltpu.make_async_remote_copy(
                src_ref=sb1_ref.at[g, pl.ds(0, 2)],
                dst_ref=rb1_ref.at[g, pl.ds(0, 2)],
                send_sem=s1b.at[g], recv_sem=r1b.at[g],
                device_id=(axes[g][0](my),),
                device_id_type=pl.DeviceIdType.MESH,
            )

        def send1a(g):
            sb1_ref[g, 2] = pp16(out_chunk(g, 2), g)
            sb1_ref[g, 3] = pp16(out_chunk(g, 3), g)
            rdma1a(g).start()

        def send1b(g):
            sb1_ref[g, 0] = pp16(out_chunk(g, 0), g)
            sb1_ref[g, 1] = pp16(out_chunk(g, 1), g)
            rdma1b(g).start()

        def combine1(g, mi):
            w_ref[g, mi] = (
                w_ref[g, mi].astype(jnp.float32)
                + rb1_ref[g, mi].astype(jnp.float32)
            ).astype(jnp.bfloat16)

        def rdma2a(g):
            return pltpu.make_async_remote_copy(
                src_ref=w_ref.at[g, pl.ds(3, 1)],
                dst_ref=rb2_ref.at[g, pl.ds(1, 1)],
                send_sem=s2.at[g], recv_sem=r2.at[g],
                device_id=(axes[g][1](my),),
                device_id_type=pl.DeviceIdType.MESH,
            )

        def rdma2b(g):
            return pltpu.make_async_remote_copy(
                src_ref=w_ref.at[g, pl.ds(2, 1)],
                dst_ref=rb2_ref.at[g, pl.ds(0, 1)],
                send_sem=s2b.at[g], recv_sem=r2b.at[g],
                device_id=(axes[g][1](my),),
                device_id_type=pl.DeviceIdType.MESH,
            )

        def do_w(g, mi):
            w_ref[g, mi] = pp16(w_chunk(g, mi), g)

        def do_p2(g):
            rdma1a(g).wait()
            combine1(g, 3)
            rdma2a(g).start()
            combine1(g, 2)
            rdma2b(g).start()

        acts = {
            3: [(send1a, 0), (do_w, 2, 3)],
            5: [(send1a, 2), (do_w, 1, 3)],
            7: [(send1b, 0), (do_w, 1, 1), (do_w, 2, 2)],
            9: [(send1a, 1), (do_w, 0, 3)],
            11: [(send1b, 1), (do_w, 0, 2), (do_w, 2, 1)],
            13: [(send1b, 2), (do_w, 0, 1), (do_w, 1, 2), (do_p2, 0)],
            15: [(do_w, 0, 0), (do_w, 1, 0), (do_w, 2, 0), (do_p2, 2)],
        }

        def a_body(t, _):
            sl = lax.rem(t, 2)
            a_dma(t).wait()
            a16_ref[pl.ds(a_row(t), a_rows), :] = (
                sta_ref[sl].astype(jnp.bfloat16))

            @pl.when(t + 2 < A_STEPS)
            def _():
                a_dma(t + 2).start()

            for trip, todo in acts.items():
                @pl.when(t == trip)
                def _(todo=todo):
                    for fn, *fa in todo:
                        fn(*fa)
            return 0

        lax.fori_loop(0, A_STEPS, a_body, 0)

        do_p2(1)
        for g in range(N_THIRDS):
            rdma1b(g).wait()
            combine1(g, 1)
            combine1(g, 0)

        def combine2(g, j):
            w_ref[g, j] = (
                w_ref[g, j].astype(jnp.float32)
                + rb2_ref[g, j].astype(jnp.float32)
            ).astype(jnp.bfloat16)

        for g in range(N_THIRDS):
            rdma2a(g).wait()
            combine2(g, 1)
            pltpu.make_async_remote_copy(
                src_ref=w_ref.at[g, pl.ds(1, 1)], dst_ref=rb3_ref.at[g],
                send_sem=s3.at[g], recv_sem=r3.at[g],
                device_id=(axes[g][2](my),),
                device_id_type=pl.DeviceIdType.MESH,
            ).start()
            rdma2b(g).wait()
            combine2(g, 0)

        for g in range(N_THIRDS):
            pltpu.make_async_remote_copy(
                src_ref=w_ref.at[g, pl.ds(1, 1)], dst_ref=rb3_ref.at[g],
                send_sem=s3.at[g], recv_sem=r3.at[g],
                device_id=(axes[g][2](my),),
                device_id_type=pl.DeviceIdType.MESH,
            ).wait()
            osl = g % 2
            if g >= 2:
                pltpu.make_async_copy(
                    ob_ref.at[osl],
                    out_hbm.at[:, pl.ds((g - 2) * nw, nw)],
                    out_sems.at[osl],
                ).wait()
            ob_ref[osl] = (
                w_ref[g, 0].astype(jnp.float32)
                + rb3_ref[g, 0].astype(jnp.float32)
            )
            pltpu.make_async_copy(
                ob_ref.at[osl], out_hbm.at[:, pl.ds(g * nw, nw)],
                out_sems.at[osl],
            ).start()
        for g in (1, 2):
            pltpu.make_async_copy(
                ob_ref.at[g % 2], out_hbm.at[:, pl.ds(g * nw, nw)],
                out_sems.at[g % 2],
            ).wait()

        for g in range(N_THIRDS):
            pl.semaphore_signal(
                exit_sem, inc=1,
                device_id=(axes[g][0](my),),
                device_id_type=pl.DeviceIdType.MESH,
            )
        pl.semaphore_wait(exit_sem, 3)

    return pl.pallas_call(
        body,
        out_shape=jax.ShapeDtypeStruct((M_CHUNK, n), jnp.float32),
        in_specs=[
            pl.BlockSpec(memory_space=pl.ANY),
            pl.BlockSpec(memory_space=pl.ANY),
        ],
        out_specs=pl.BlockSpec(memory_space=pl.ANY),
        scratch_shapes=[
            pltpu.VMEM((m, k), jnp.bfloat16),
            pltpu.VMEM((k, n), jnp.bfloat16),
            pltpu.VMEM((2, M_CHUNK // 2, k), jnp.float32),
            pltpu.VMEM((2, k // B_STEPS, n), jnp.float32),
            pltpu.VMEM((N_THIRDS, 4, M_CHUNK, nw), jnp.bfloat16),
            pltpu.VMEM((N_THIRDS, 4, M_CHUNK, nw), jnp.bfloat16),
            pltpu.VMEM((N_THIRDS, 4, M_CHUNK, nw), jnp.bfloat16),
            pltpu.VMEM((N_THIRDS, 2, M_CHUNK, nw), jnp.bfloat16),
            pltpu.VMEM((N_THIRDS, 1, M_CHUNK, nw), jnp.bfloat16),
            pltpu.VMEM((2, M_CHUNK, nw), jnp.float32),
            pltpu.SemaphoreType.DMA((2,)),
            pltpu.SemaphoreType.DMA((2,)),
            pltpu.SemaphoreType.DMA((2,)),
            pltpu.SemaphoreType.DMA((N_THIRDS,)),
            pltpu.SemaphoreType.DMA((N_THIRDS,)),
            pltpu.SemaphoreType.DMA((N_THIRDS,)),
            pltpu.SemaphoreType.DMA((N_THIRDS,)),
            pltpu.SemaphoreType.DMA((N_THIRDS,)),
            pltpu.SemaphoreType.DMA((N_THIRDS,)),
            pltpu.SemaphoreType.DMA((N_THIRDS,)),
            pltpu.SemaphoreType.DMA((N_THIRDS,)),
            pltpu.SemaphoreType.DMA((N_THIRDS,)),
            pltpu.SemaphoreType.DMA((N_THIRDS,)),
            pltpu.SemaphoreType.REGULAR,
        ],
        compiler_params=pltpu.CompilerParams(
            collective_id=0,
            vmem_limit_bytes=100 * 1024 * 1024,
        ),
    )(A, B)


# device time: 105652 ns/iter; 1.0002x vs baseline; 1.0002x over previous
import jax
import jax.numpy as jnp
from jax import lax
from jax.experimental import pallas as pl
from jax.experimental.pallas import tpu as pltpu

N_DEV = 8
M = 3072
M_CHUNK = M // N_DEV
N_THIRDS = 3

A_STEPS = 16
B_STEPS = 16


def kernel(A, B):
    m, k = A.shape
    k2, n = B.shape
    assert m == M and k == k2
    nw = n // N_THIRDS
    a_rows = M_CHUNK // 2
    b_rows = k // B_STEPS

    def body(a_hbm, b_hbm, out_hbm, a16_ref, b16_ref, sta_ref, stb_ref,
             sb1_ref, rb1_ref, w_ref, rb2_ref, rb3_ref, ob_ref,
             cp_sems_a, cp_sems_b, out_sems,
             s1, r1, s1b, r1b, s2, r2, s2b, r2b, s3, r3, exit_sem):
        my = lax.axis_index("i")

        def fx(p):
            return p + 1 - 2 * lax.rem(p, 2)

        def fy(p):
            p4 = lax.rem(p, 4)
            return p - p4 + 3 - p4

        def fz(p):
            return lax.rem(p + 4, N_DEV)

        def F(p, mask):
            bx = mask // 4
            by = lax.rem(mask, 4) // 2
            bz = lax.rem(mask, 2)
            p = bz * fz(p) + (1 - bz) * p
            p = by * fy(p) + (1 - by) * p
            p = bx * fx(p) + (1 - bx) * p
            return p

        axes = [(fx, fy, fz), (fy, fz, fx), (fz, fx, fy)]

        def w_chunk(g, mi):
            _, F1, F2 = axes[g]
            c = my
            if mi & 1:
                c = F2(c)
            if mi & 2:
                c = F1(c)
            return c

        def out_chunk(g, mi):
            return axes[g][0](w_chunk(g, mi))

        barrier_sem = pltpu.get_barrier_semaphore()
        for g in range(N_THIRDS):
            pl.semaphore_signal(
                barrier_sem, inc=1,
                device_id=(axes[g][0](my),),
                device_id_type=pl.DeviceIdType.MESH,
            )
        pl.semaphore_wait(barrier_sem, 3)

        def b_dma(j):
            return pltpu.make_async_copy(
                b_hbm.at[pl.ds(j * b_rows, b_rows), :],
                stb_ref.at[lax.rem(j, 2)], cp_sems_b.at[lax.rem(j, 2)],
            )

        def a_chunk_of(t):
            return F(my, 7 - t // 2)

        def a_row(t):
            return a_chunk_of(t) * M_CHUNK + lax.rem(t, 2) * a_rows

        def a_dma(t):
            return pltpu.make_async_copy(
                a_hbm.at[pl.ds(a_row(t), a_rows), :],
                sta_ref.at[lax.rem(t, 2)], cp_sems_a.at[lax.rem(t, 2)],
            )

        b_dma(0).start()
        b_dma(1).start()
        a_dma(0).start()
        a_dma(1).start()

        def b_body(j, _):
            sl = lax.rem(j, 2)
            b_dma(j).wait()
            b16_ref[pl.ds(j * b_rows, b_rows), :] = (
                stb_ref[sl].astype(jnp.bfloat16))

            @pl.when(j + 2 < B_STEPS)
            def _():
                b_dma(j + 2).start()
            return 0

        lax.fori_loop(0, B_STEPS, b_body, 0)

        def rows(c):
            return pl.ds(c * M_CHUNK, M_CHUNK)

        def pp16(c, g):
            return jnp.dot(
                a16_ref[rows(c), :], b16_ref[:, g * nw:(g + 1) * nw],
                preferred_element_type=jnp.float32,
            ).astype(jnp.bfloat16)

        def rdma1a(g):
            return pltpu.make_async_remote_copy(
                src_ref=sb1_ref.at[g, pl.ds(2, 2)],
                dst_ref=rb1_ref.at[g, pl.ds(2, 2)],
                send_sem=s1.at[g], recv_sem=r1.at[g],
                device_id=(axes[g][0](my),),
                device_id_type=pl.DeviceIdType.MESH,
            )

        def rdma1b(g):
            return pltpu.make_async_remote_copy(
                src_ref=sb1_ref.at[g, pl.ds(0, 2)],
                dst_ref=rb1_ref.at[g, pl.ds(0, 2)],
                send_sem=s1b.at[g], recv_sem=r1b.at[g],
                device_id=(axes[g][0](my),),
                device_id_type=pl.DeviceIdType.MESH,
            )

        def send1a(g):
            sb1_ref[g, 2] = pp16(out_chunk(g, 2), g)
            sb1_ref[g, 3] = pp16(out_chunk(g, 3), g)
            rdma1a(g).start()

        def send1b(g):
            sb1_ref[g, 0] = pp16(out_chunk(g, 0), g)
            sb1_ref[g, 1] = pp16(out_chunk(g, 1), g)
            rdma1b(g).start()

        def combine1(g, mi):
            w_ref[g, mi] = (
                w_ref[g, mi].astype(jnp.float32)
                + rb1_ref[g, mi].astype(jnp.float32)
            ).astype(jnp.bfloat16)

        def rdma2a(g):
            return pltpu.make_async_remote_copy(
                src_ref=w_ref.at[g, pl.ds(3, 1)],
                dst_ref=rb2_ref.at[g, pl.ds(1, 1)],
                send_sem=s2.at[g], recv_sem=r2.at[g],
                device_id=(axes[g][1](my),),
                device_id_type=pl.DeviceIdType.MESH,
            )

        def rdma2b(g):
            return pltpu.make_async_remote_copy(
                src_ref=w_ref.at[g, pl.ds(2, 1)],
                dst_ref=rb2_ref.at[g, pl.ds(0, 1)],
                send_sem=s2b.at[g], recv_sem=r2b.at[g],
                device_id=(axes[g][1](my),),
                device_id_type=pl.DeviceIdType.MESH,
            )

        def do_w(g, mi):
            w_ref[g, mi] = pp16(w_chunk(g, mi), g)

        def do_p2(g):
            rdma1a(g).wait()
            combine1(g, 3)
            rdma2a(g).start()
            combine1(g, 2)
            rdma2b(g).start()

        acts = {
            3: [(send1a, 0), (do_w, 2, 3)],
            5: [(send1a, 2), (do_w, 1, 3)],
            7: [(send1b, 0), (do_w, 1, 1), (do_w, 2, 2)],
            9: [(send1a, 1), (do_w, 0, 3)],
            11: [(send1b, 1), (do_w, 0, 2), (do_w, 2, 1)],
            13: [(send1b, 2), (do_w, 0, 1), (do_w, 1, 2)],
            15: [(do_w, 0, 0), (do_w, 1, 0), (do_w, 2, 0)],
        }

        def a_body(t, _):
            sl = lax.rem(t, 2)
            a_dma(t).wait()
            a16_ref[pl.ds(a_row(t), a_rows), :] = (
                sta_ref[sl].astype(jnp.bfloat16))

            @pl.when(t + 2 < A_STEPS)
            def _():
                a_dma(t + 2).start()

            for trip, todo in acts.items():
                @pl.when(t == trip)
                def _(todo=todo):
                    for fn, *fa in todo:
                        fn(*fa)
            return 0

        lax.fori_loop(0, A_STEPS, a_body, 0)

        do_p2(0)
        do_p2(2)
        do_p2(1)
        for g in range(N_THIRDS):
            rdma1b(g).wait()
            combine1(g, 1)
            combine1(g, 0)

        def combine2(g, j):
            w_ref[g, j] = (
                w_ref[g, j].astype(jnp.float32)
                + rb2_ref[g, j].astype(jnp.float32)
            ).astype(jnp.bfloat16)

        for g in range(N_THIRDS):
            rdma2a(g).wait()
            combine2(g, 1)
            pltpu.make_async_remote_copy(
                src_ref=w_ref.at[g, pl.ds(1, 1)], dst_ref=rb3_ref.at[g],
                send_sem=s3.at[g], recv_sem=r3.at[g],
                device_id=(axes[g][2](my),),
                device_id_type=pl.DeviceIdType.MESH,
            ).start()
            rdma2b(g).wait()
            combine2(g, 0)

        for g in range(N_THIRDS):
            pltpu.make_async_remote_copy(
                src_ref=w_ref.at[g, pl.ds(1, 1)], dst_ref=rb3_ref.at[g],
                send_sem=s3.at[g], recv_sem=r3.at[g],
                device_id=(axes[g][2](my),),
                device_id_type=pl.DeviceIdType.MESH,
            ).wait()
            osl = g % 2
            if g >= 2:
                pltpu.make_async_copy(
                    ob_ref.at[osl],
                    out_hbm.at[:, pl.ds((g - 2) * nw, nw)],
                    out_sems.at[osl],
                ).wait()
            ob_ref[osl] = (
                w_ref[g, 0].astype(jnp.float32)
                + rb3_ref[g, 0].astype(jnp.float32)
            )
            pltpu.make_async_copy(
                ob_ref.at[osl], out_hbm.at[:, pl.ds(g * nw, nw)],
                out_sems.at[osl],
            ).start()
        for g in (1, 2):
            pltpu.make_async_copy(
                ob_ref.at[g % 2], out_hbm.at[:, pl.ds(g * nw, nw)],
                out_sems.at[g % 2],
            ).wait()

        for g in range(N_THIRDS):
            pl.semaphore_signal(
                exit_sem, inc=1,
                device_id=(axes[g][0](my),),
                device_id_type=pl.DeviceIdType.MESH,
            )
        pl.semaphore_wait(exit_sem, 3)

    return pl.pallas_call(
        body,
        out_shape=jax.ShapeDtypeStruct((M_CHUNK, n), jnp.float32),
        in_specs=[
            pl.BlockSpec(memory_space=pl.ANY),
            pl.BlockSpec(memory_space=pl.ANY),
        ],
        out_specs=pl.BlockSpec(memory_space=pl.ANY),
        scratch_shapes=[
            pltpu.VMEM((m, k), jnp.bfloat16),
            pltpu.VMEM((k, n), jnp.bfloat16),
            pltpu.VMEM((2, M_CHUNK // 2, k), jnp.float32),
            pltpu.VMEM((2, k // B_STEPS, n), jnp.float32),
            pltpu.VMEM((N_THIRDS, 4, M_CHUNK, nw), jnp.bfloat16),
            pltpu.VMEM((N_THIRDS, 4, M_CHUNK, nw), jnp.bfloat16),
            pltpu.VMEM((N_THIRDS, 4, M_CHUNK, nw), jnp.bfloat16),
            pltpu.VMEM((N_THIRDS, 2, M_CHUNK, nw), jnp.bfloat16),
            pltpu.VMEM((N_THIRDS, 1, M_CHUNK, nw), jnp.bfloat16),
            pltpu.VMEM((2, M_CHUNK, nw), jnp.float32),
            pltpu.SemaphoreType.DMA((2,)),
            pltpu.SemaphoreType.DMA((2,)),
            pltpu.SemaphoreType.DMA((2,)),
            pltpu.SemaphoreType.DMA((N_THIRDS,)),
            pltpu.SemaphoreType.DMA((N_THIRDS,)),
            pltpu.SemaphoreType.DMA((N_THIRDS,)),
            pltpu.SemaphoreType.DMA((N_THIRDS,)),
            pltpu.SemaphoreType.DMA((N_THIRDS,)),
            pltpu.SemaphoreType.DMA((N_THIRDS,)),
            pltpu.SemaphoreType.DMA((N_THIRDS,)),
            pltpu.SemaphoreType.DMA((N_THIRDS,)),
            pltpu.SemaphoreType.DMA((N_THIRDS,)),
            pltpu.SemaphoreType.DMA((N_THIRDS,)),
            pltpu.SemaphoreType.REGULAR,
        ],
        compiler_params=pltpu.CompilerParams(
            collective_id=0,
            vmem_limit_bytes=100 * 1024 * 1024,
        ),
    )(A, B)
